# baseline (device time: 535169 ns/iter reference)
import jax
import jax.numpy as jnp
from jax import lax
from jax.experimental import pallas as pl
from jax.experimental.pallas import tpu as pltpu

N_DEV = 4
N_CHUNK = 4
BN = 512


def _make_fused_body(m, k_sh, n):
    h = k_sh // 2
    cw_ = n // N_CHUNK
    n_sub = n // BN
    sub_per_chunk = cw_ // BN

    def body(x_ref, w_ref, s_ref, y_ref, wg_ref,
             xv, wv, yv, local_sems, wv_sems, yv_sems,
             sx_sems, rx_sems, sw_sems, rw_sems):
        me = lax.axis_index("i")
        a = jnp.bitwise_xor(me, 1)
        b = 3 - me

        def start(src, dst, ssem, rsem, dev):
            c = pltpu.make_async_remote_copy(
                src_ref=src, dst_ref=dst, send_sem=ssem, recv_sem=rsem,
                device_id=(dev,), device_id_type=pl.DeviceIdType.MESH)
            c.start()
            return c

        def recv_wait(dst, dummy_src, rsem):
            pltpu.make_async_remote_copy(
                src_ref=dummy_src, dst_ref=dst,
                send_sem=rsem, recv_sem=rsem,
                device_id=(me,),
                device_id_type=pl.DeviceIdType.MESH).wait_recv()

        def x_slot(q, half):
            return xv.at[:, pl.ds(q * k_sh + half * h, h)]

        def x_half(half):
            return x_ref.at[:, pl.ds(half * h, h)]

        def w_slot(q, half, c):
            return wg_ref.at[pl.ds(q * k_sh + half * h, h),
                             pl.ds(c * cw_, cw_)]

        def w_half(half, c):
            return w_ref.at[pl.ds(half * h, h), pl.ds(c * cw_, cw_)]

        barrier = pltpu.get_barrier_semaphore()
        for p in (a, b):
            pl.semaphore_signal(barrier, inc=1, device_id=(p,),
                                device_id_type=pl.DeviceIdType.MESH)
        pl.semaphore_wait(barrier, 2)

        cx = pltpu.make_async_copy(
            x_ref, xv.at[:, pl.ds(me * k_sh, k_sh)], local_sems.at[0])
        cw = pltpu.make_async_copy(
            w_ref, wg_ref.at[pl.ds(me * k_sh, k_sh), :], local_sems.at[1])
        cx.start()
        cw.start()

        sends = []

        sends.append(start(x_half(0), x_slot(me, 0), sx_sems.at[0], rx_sems.at[0], a))
        sends.append(start(x_half(1), x_slot(me, 1), sx_sems.at[1], rx_sems.at[1], b))
        sends.append(start(x_half(0), x_slot(me, 0), sx_sems.at[2], rx_sems.at[2], b))
        sends.append(start(x_half(1), x_slot(me, 1), sx_sems.at[4], rx_sems.at[4], a))
        recv_wait(x_slot(a, 0), x_half(0), rx_sems.at[0])
        sends.append(start(x_slot(a, 0), x_slot(a, 0), sx_sems.at[3], rx_sems.at[3], b))
        recv_wait(x_slot(b, 1), x_half(1), rx_sems.at[1])
        sends.append(start(x_slot(b, 1), x_slot(b, 1), sx_sems.at[5], rx_sems.at[5], a))

        for c in range(N_CHUNK):
            sends.append(start(w_half(0, c), w_slot(me, 0, c), sw_sems.at[c, 0], rw_sems.at[c, 0], a))
            sends.append(start(w_half(1, c), w_slot(me, 1, c), sw_sems.at[c, 1], rw_sems.at[c, 1], b))
            sends.append(start(w_half(0, c), w_slot(me, 0, c), sw_sems.at[c, 2], rw_sems.at[c, 2], b))
            sends.append(start(w_half(1, c), w_slot(me, 1, c), sw_sems.at[c, 4], rw_sems.at[c, 4], a))

        recv_wait(x_slot(b, 0), x_half(0), rx_sems.at[2])
        recv_wait(x_slot(jnp.bitwise_xor(b, 1), 0), x_half(0), rx_sems.at[3])
        recv_wait(x_slot(a, 1), x_half(1), rx_sems.at[4])
        recv_wait(x_slot(3 - a, 1), x_half(1), rx_sems.at[5])
        cx.wait()

        def w_fwd(c):
            recv_wait(w_slot(a, 0, c), w_half(0, c), rw_sems.at[c, 0])
            start(w_slot(a, 0, c), w_slot(a, 0, c),
                  sw_sems.at[c, 3], rw_sems.at[c, 3], b)
            recv_wait(w_slot(b, 1, c), w_half(1, c), rw_sems.at[c, 1])
            start(w_slot(b, 1, c), w_slot(b, 1, c),
                  sw_sems.at[c, 5], rw_sems.at[c, 5], a)

        def w_complete(c):
            recv_wait(w_slot(b, 0, c), w_half(0, c), rw_sems.at[c, 2])
            recv_wait(w_slot(jnp.bitwise_xor(b, 1), 0, c), w_half(0, c), rw_sems.at[c, 3])
            recv_wait(w_slot(a, 1, c), w_half(1, c), rw_sems.at[c, 4])
            recv_wait(w_slot(3 - a, 1, c), w_half(1, c), rw_sems.at[c, 5])

        w_fwd(0)
        w_complete(0)
        cw.wait()

        def wv_dma(g, buf):
            return pltpu.make_async_copy(
                wg_ref.at[:, pl.ds(g * BN, BN)], wv.at[buf], wv_sems.at[buf])

        def yv_dma(g, buf):
            return pltpu.make_async_copy(
                yv.at[buf], y_ref.at[:, pl.ds(g * BN, BN)], yv_sems.at[buf])

        scale = s_ref[0, 0]

        def dot_into(buf):
            yv[buf] = lax.dot_general(
                xv[...], wv[buf], (((1,), (0,)), ((), ())),
                preferred_element_type=jnp.float32) * scale

        def pair_body(c, p):
            g0 = c * sub_per_chunk + 2 * p
            wv_dma(g0 + 1, 1).start()
            wv_dma(g0, 0).wait()

            @pl.when(g0 >= 2)
            def _():
                yv_dma(g0 - 2, 0).wait()

            dot_into(0)
            yv_dma(g0, 0).start()

            @pl.when(p < sub_per_chunk // 2 - 1)
            def _():
                wv_dma(g0 + 2, 0).start()

            wv_dma(g0 + 1, 1).wait()

            @pl.when(g0 + 1 >= 2)
            def _():
                yv_dma(g0 - 1, 1).wait()

            dot_into(1)
            yv_dma(g0 + 1, 1).start()

        def chunk_body(c, carry):
            @pl.when(c < N_CHUNK - 1)
            def _():
                w_fwd(c + 1)

            def _pair(p, cr):
                pair_body(c, p)
                return cr

            lax.fori_loop(0, sub_per_chunk // 2, _pair, 0)

            @pl.when(c < N_CHUNK - 1)
            def _():
                w_complete(c + 1)
                wv_dma((c + 1) * sub_per_chunk, 0).start()

            return carry

        wv_dma(0, 0).start()
        lax.fori_loop(0, N_CHUNK, chunk_body, 0)

        yv_dma(n_sub - 2, 0).wait()
        yv_dma(n_sub - 1, 1).wait()
        for s in sends:
            s.wait_send()
        for c in range(N_CHUNK):
            for i in (3, 5):
                pltpu.make_async_remote_copy(
                    src_ref=w_half(0, 0), dst_ref=w_slot(me, 0, 0),
                    send_sem=sw_sems.at[c, i], recv_sem=rw_sems.at[c, i],
                    device_id=(me,),
                    device_id_type=pl.DeviceIdType.MESH).wait_send()

    return body


def kernel(x, w_mat, scale_x, scale_w):
    m, k_sh = x.shape
    _, n = w_mat.shape
    k = k_sh * N_DEV

    x8 = x.astype(jnp.float8_e5m2)
    w8 = w_mat.astype(jnp.float8_e5m2)
    scale = (scale_x[0] * scale_w[0]).reshape(1, 1)

    y, _ = pl.pallas_call(
        _make_fused_body(m, k_sh, n),
        out_shape=[
            jax.ShapeDtypeStruct((m, n), jnp.float32),
            jax.ShapeDtypeStruct((k, n), jnp.float8_e5m2),
        ],
        in_specs=[
            pl.BlockSpec(memory_space=pl.ANY),
            pl.BlockSpec(memory_space=pl.ANY),
            pl.BlockSpec(memory_space=pltpu.SMEM),
        ],
        out_specs=[
            pl.BlockSpec(memory_space=pl.ANY),
            pl.BlockSpec(memory_space=pl.ANY),
        ],
        scratch_shapes=[
            pltpu.VMEM((m, k), jnp.float8_e5m2),
            pltpu.VMEM((2, k, BN), jnp.float8_e5m2),
            pltpu.VMEM((2, m, BN), jnp.float32),
            pltpu.SemaphoreType.DMA((2,)),
            pltpu.SemaphoreType.DMA((2,)),
            pltpu.SemaphoreType.DMA((2,)),
            pltpu.SemaphoreType.DMA((6,)),
            pltpu.SemaphoreType.DMA((6,)),
            pltpu.SemaphoreType.DMA((N_CHUNK, 6)),
            pltpu.SemaphoreType.DMA((N_CHUNK, 6)),
        ],
        compiler_params=pltpu.CompilerParams(
            collective_id=0, vmem_limit_bytes=56 * 1024 * 1024),
    )(x8, w8, scale)
    return y
